# baseline (speedup 1.0000x reference)
import functools
import jax
import jax.numpy as jnp
from jax import lax
from jax.experimental import pallas as pl
from jax.experimental.pallas import tpu as pltpu
from jax.experimental.pallas import tpu_sc as plsc

DAMPING = 0.1
P = 4
NBUF = 2
LANES = 16


def _sc_body(x_hbm, pos_hbm, o_hbm, pos_t, ib0, ib1,
             psem, isem0, isem1, osem0, osem1):
    B, R, C, E = x_hbm.shape
    NCH = R // P
    w = lax.axis_index("s") * 2 + lax.axis_index("c")

    pltpu.make_async_copy(pos_hbm, pos_t, psem).start()
    pltpu.make_async_copy(pos_hbm, pos_t, psem).wait()

    ibufs = (ib0, ib1)
    isems = (isem0, isem1)
    osems = (osem0, osem1)

    def in_copy(g, k):
        return pltpu.make_async_copy(
            x_hbm.at[w, pl.ds(g * P, P)], ibufs[k], isems[k])

    def out_copy(g, k):
        return pltpu.make_async_copy(
            ibufs[k], o_hbm.at[w, pl.ds(g * P, P)], osems[k])

    def compute(g, k):
        ib = ibufs[k]
        for p in range(P):
            r = g * P + p
            for eb in range(E // LANES):
                pv = pos_t[r, pl.ds(eb * LANES, LANES)] * DAMPING
                for c in range(C):
                    plsc.addupdate(ib.at[p, c, pl.ds(eb * LANES, LANES)], pv)

    in_copy(0, 0).start()
    in_copy(0, 0).wait()
    compute(0, 0)
    out_copy(0, 0).start()
    in_copy(1, 1).start()

    def mid(sstep, carry):
        j1 = 2 * sstep + 1
        in_copy(j1, 1).wait()
        compute(j1, 1)
        out_copy(j1, 1).start()
        out_copy(j1 - 1, 0).wait()
        in_copy(j1 + 1, 0).start()
        j2 = j1 + 1
        in_copy(j2, 0).wait()
        compute(j2, 0)
        out_copy(j2, 0).start()
        out_copy(j2 - 1, 1).wait()
        in_copy(j2 + 1, 1).start()
        return carry

    lax.fori_loop(0, (NCH - 2) // 2, mid, 0)

    in_copy(NCH - 1, 1).wait()
    compute(NCH - 1, 1)
    out_copy(NCH - 1, 1).start()
    out_copy(NCH - 2, 0).wait()
    out_copy(NCH - 1, 1).wait()


def kernel(x, pos_table):
    B, R, C, E = x.shape
    mesh = plsc.VectorSubcoreMesh(core_axis_name="c", subcore_axis_name="s")
    run = functools.partial(
        pl.kernel,
        mesh=mesh,
        out_type=jax.ShapeDtypeStruct(x.shape, x.dtype),
        scratch_types=[
            pltpu.VMEM((R, E), jnp.float32),
            pltpu.VMEM((P, C, E), jnp.float32),
            pltpu.VMEM((P, C, E), jnp.float32),
            pltpu.SemaphoreType.DMA,
            pltpu.SemaphoreType.DMA,
            pltpu.SemaphoreType.DMA,
            pltpu.SemaphoreType.DMA,
            pltpu.SemaphoreType.DMA,
        ],
    )(_sc_body)
    return run(x, pos_table)
